# baseline (speedup 1.0000x reference)
import functools

import jax
import jax.numpy as jnp
from jax import lax
from jax.experimental import pallas as pl
from jax.experimental.pallas import tpu as pltpu
from jax.experimental.pallas import tpu_sc as plsc

VOCAB = 1000000
DIM = 64
BATCH = 4096
HIST = 200

NC, NS = 2, 16
NW = NC * NS
BW = BATCH // NW
DT = DIM // 8


def _embed_body(idx_hbm, table_hbm, out_hbm, idx_v, rows0, rows1, tb0, tb1,
                sem_g0, sem_g1, sem_o0, sem_o1):
    wid = lax.axis_index("s") * NC + lax.axis_index("c")
    b0 = wid * BW
    pltpu.sync_copy(idx_hbm.at[:, pl.ds(b0, BW)], idx_v)

    rows = (rows0, rows1)
    tbs = (tb0, tb1)
    sem_g = (sem_g0, sem_g1)
    sem_o = (sem_o0, sem_o1)

    lane = lax.iota(jnp.int32, 16)

    def start_gather(h, b):
        pltpu.async_copy(table_hbm.at[idx_v.at[h]], rows[b], sem_g[b])

    def start_out(h, b):
        pltpu.async_copy(tbs[b], out_hbm.at[h, :, wid], sem_o[b])

    def wait_gather(b):
        pltpu.make_async_copy(table_hbm.at[pl.ds(0, BW)], rows[b],
                              sem_g[b]).wait()

    def wait_out(b):
        pltpu.make_async_copy(tbs[b], out_hbm.at[0, :, wid], sem_o[b]).wait()

    def transpose(b):
        def col(d, _):
            dt = d // 8
            ds = d % 8
            dvec = jnp.full((16,), 0, jnp.int32) + d
            for g in range(BW // 16):
                bvec = lane + (16 * g)
                v = plsc.load_gather(rows[b], [bvec, dvec])
                tbs[b][dt, ds, pl.ds(16 * g, 16)] = v
            return _

        lax.fori_loop(0, DIM, col, 0)

    start_gather(0, 0)
    start_gather(1, 1)

    def half(h, b):
        wait_gather(b)

        @pl.when(h >= 2)
        def _w():
            wait_out(b)

        transpose(b)

        @pl.when(h + 2 < HIST)
        def _g():
            start_gather(h + 2, b)

        start_out(h, b)

    def step(i, _):
        half(2 * i, 0)
        half(2 * i + 1, 1)
        return _

    lax.fori_loop(0, HIST // 2, step, 0)
    wait_out(0)
    wait_out(1)


@jax.jit
def _embed(idx_t, table):
    mesh = plsc.VectorSubcoreMesh(core_axis_name="c", subcore_axis_name="s")
    return pl.kernel(
        _embed_body,
        out_type=jax.ShapeDtypeStruct((HIST, DT, NW, 8, BW), jnp.float32),
        mesh=mesh,
        compiler_params=pltpu.CompilerParams(use_tc_tiling_on_sc=False,
                                            needs_layout_passes=False),
        scratch_types=[
            pltpu.VMEM((HIST, BW), jnp.int32),
            pltpu.VMEM((BW, DIM), jnp.float32),
            pltpu.VMEM((BW, DIM), jnp.float32),
            pltpu.VMEM((DT, 8, BW), jnp.float32),
            pltpu.VMEM((DT, 8, BW), jnp.float32),
            pltpu.SemaphoreType.DMA,
            pltpu.SemaphoreType.DMA,
            pltpu.SemaphoreType.DMA,
            pltpu.SemaphoreType.DMA,
        ],
    )(idx_t, table)


def kernel(inp, embeddings):
    idx_t = inp.T.astype(jnp.int32)
    out5 = _embed(idx_t, embeddings)
    return out5.transpose(2, 4, 0, 1, 3).reshape(BATCH, HIST, DIM)
